# Initial kernel scaffold; baseline (speedup 1.0000x reference)
import jax
import jax.numpy as jnp
from jax import lax
from jax.experimental import pallas as pl
from jax.experimental.pallas import tpu as pltpu
from jax.experimental.pallas import tpu_sc as plsc

N_NODES = 50000
DIM = 32
N_EDGES = 1600000
N_LAYERS = 3

NC, NS = 2, 16
NW = NC * NS

NP = 50048
DUMMY = 2176
NACC = NP + DUMMY

EPT = 51200
E_PAD = EPT * NW
CHUNK = 2048
N_CHUNKS = EPT // CHUNK
JSTEPS = CHUNK // 128
E2D = E_PAD // 128

ZROWS = NACC // NS
DROWS = NP // NS
DHALF = DROWS // 2

_MESH = plsc.VectorSubcoreMesh(core_axis_name="c", subcore_axis_name="s")


def _deg_body(colp, degp, colbuf, ones1, zbuf, acc):
    c = lax.axis_index("c")
    s = lax.axis_index("s")
    w = c * NS + s

    def fill(i, _):
        ones1[pl.ds(i * 16, 16)] = jnp.full((16,), 1.0, jnp.float32)
        return 0
    lax.fori_loop(0, 8, fill, 0)

    def fillz(i, _):
        zbuf[pl.ds(i * 16, 16)] = jnp.zeros((16,), jnp.float32)
        return 0
    lax.fori_loop(0, ZROWS // 16, fillz, 0)
    pltpu.sync_copy(zbuf, acc.at[pl.ds(s * ZROWS, ZROWS)])
    plsc.subcore_barrier()

    eb = w * (EPT // 128)

    def chunk(i, _):
        pltpu.sync_copy(colp.at[pl.ds(eb + i * JSTEPS, JSTEPS)], colbuf)
        for j in range(JSTEPS):
            pltpu.sync_copy(ones1, acc.at[colbuf.at[j]], add=True)
        return 0
    lax.fori_loop(0, N_CHUNKS, chunk, 0)
    plsc.subcore_barrier()

    pltpu.sync_copy(acc.at[pl.ds(s * DROWS, DROWS)], zbuf.at[pl.ds(0, DROWS)])
    pltpu.sync_copy(zbuf.at[pl.ds(0, DROWS)], degp.at[c, pl.ds(s * DROWS, DROWS)])


_deg_kernel = pl.kernel(
    _deg_body,
    out_type=jax.ShapeDtypeStruct((NC, NP), jnp.float32),
    mesh=_MESH,
    scratch_types=[
        pltpu.VMEM((JSTEPS, 128), jnp.int32),
        pltpu.VMEM((128,), jnp.float32),
        pltpu.VMEM((ZROWS,), jnp.float32),
        pltpu.VMEM_SHARED((NACC,), jnp.float32),
    ],
)


def _layer_body(y, rowp, colp, part, rowbuf, colbuf, rows, acc, sem):
    c = lax.axis_index("c")
    s = lax.axis_index("s")
    w = c * NS + s

    def fillz(i, _):
        rows[i, pl.ds(0, 16)] = jnp.zeros((16,), jnp.float32)
        rows[i, pl.ds(16, 16)] = jnp.zeros((16,), jnp.float32)
        return 0
    lax.fori_loop(0, ZROWS // 2, fillz, 0)
    for k in range(2):
        pltpu.sync_copy(rows.at[pl.ds(0, ZROWS // 2)],
                        acc.at[pl.ds(s * ZROWS + k * (ZROWS // 2), ZROWS // 2)])
    plsc.subcore_barrier()

    eb = w * (EPT // 128)

    def chunk(i, _):
        pltpu.sync_copy(rowp.at[pl.ds(eb + i * JSTEPS, JSTEPS)], rowbuf)
        pltpu.sync_copy(colp.at[pl.ds(eb + i * JSTEPS, JSTEPS)], colbuf)
        descs = [
            pltpu.async_copy(y.at[rowbuf.at[j]],
                             rows.at[pl.ds(j * 128, 128)], sem)
            for j in range(JSTEPS)
        ]
        for d in descs:
            d.wait()
        for j in range(JSTEPS):
            pltpu.sync_copy(rows.at[pl.ds(j * 128, 128)],
                            acc.at[colbuf.at[j]], add=True)
        return 0
    lax.fori_loop(0, N_CHUNKS, chunk, 0)
    plsc.subcore_barrier()

    for k in range(2):
        base = s * DROWS + k * DHALF
        pltpu.sync_copy(acc.at[pl.ds(base, DHALF)], rows.at[pl.ds(0, DHALF)])
        pltpu.sync_copy(rows.at[pl.ds(0, DHALF)], part.at[c, pl.ds(base, DHALF)])


_layer_kernel = pl.kernel(
    _layer_body,
    out_type=jax.ShapeDtypeStruct((NC, NP, DIM), jnp.float32),
    mesh=_MESH,
    scratch_types=[
        pltpu.VMEM((JSTEPS, 128), jnp.int32),
        pltpu.VMEM((JSTEPS, 128), jnp.int32),
        pltpu.VMEM((CHUNK, DIM), jnp.float32),
        pltpu.VMEM_SHARED((NACC, DIM), jnp.float32),
        pltpu.SemaphoreType.DMA,
    ],
)


BR = NP // 16


def _tc_init_body(degp_ref, w_ref, disb_ref, y0_ref):
    deg = degp_ref[0] + degp_ref[1]
    dis = jnp.where(deg > 0, lax.rsqrt(deg), 0.0)
    disb = jnp.broadcast_to(dis, (BR, DIM))
    disb_ref[...] = disb
    y0_ref[...] = disb * w_ref[...]


_tc_init = pl.pallas_call(
    _tc_init_body,
    grid=(16,),
    in_specs=[
        pl.BlockSpec((2, BR, 1), lambda i: (0, i, 0)),
        pl.BlockSpec((BR, DIM), lambda i: (i, 0)),
    ],
    out_specs=[
        pl.BlockSpec((BR, DIM), lambda i: (i, 0)),
        pl.BlockSpec((BR, DIM), lambda i: (i, 0)),
    ],
    out_shape=[
        jax.ShapeDtypeStruct((NP, DIM), jnp.float32),
        jax.ShapeDtypeStruct((NP, DIM), jnp.float32),
    ],
)


def _tc_layer_body(p_ref, disb_ref, acc_ref, y_ref, accout_ref):
    a = p_ref[0] + p_ref[1]
    x = disb_ref[...] * a
    accout_ref[...] = acc_ref[...] + x
    y_ref[...] = disb_ref[...] * x


_tc_layer = pl.pallas_call(
    _tc_layer_body,
    grid=(16,),
    in_specs=[
        pl.BlockSpec((2, BR, DIM), lambda i: (0, i, 0)),
        pl.BlockSpec((BR, DIM), lambda i: (i, 0)),
        pl.BlockSpec((BR, DIM), lambda i: (i, 0)),
    ],
    out_specs=[
        pl.BlockSpec((BR, DIM), lambda i: (i, 0)),
        pl.BlockSpec((BR, DIM), lambda i: (i, 0)),
    ],
    out_shape=[
        jax.ShapeDtypeStruct((NP, DIM), jnp.float32),
        jax.ShapeDtypeStruct((NP, DIM), jnp.float32),
    ],
)


def _tc_final_body(p_ref, disb_ref, acc_ref, out_ref):
    a = p_ref[0] + p_ref[1]
    x = disb_ref[...] * a
    out_ref[...] = (acc_ref[...] + x) * (1.0 / (N_LAYERS + 1))


_tc_final = pl.pallas_call(
    _tc_final_body,
    grid=(16,),
    in_specs=[
        pl.BlockSpec((2, BR, DIM), lambda i: (0, i, 0)),
        pl.BlockSpec((BR, DIM), lambda i: (i, 0)),
        pl.BlockSpec((BR, DIM), lambda i: (i, 0)),
    ],
    out_specs=pl.BlockSpec((BR, DIM), lambda i: (i, 0)),
    out_shape=jax.ShapeDtypeStruct((NP, DIM), jnp.float32),
)


def kernel(edge_index, W):
    row = edge_index[0]
    col = edge_index[1]
    pad = E_PAD - N_EDGES
    rowp = jnp.concatenate(
        [row, jnp.zeros((pad,), jnp.int32)]).reshape(E2D, 128)
    colp = jnp.concatenate(
        [col, NP + (jnp.arange(pad, dtype=jnp.int32) % DUMMY)]).reshape(E2D, 128)
    Wp = jnp.pad(W, ((0, NP - N_NODES), (0, 0)))

    degp = _deg_kernel(colp)
    disb, y = _tc_init(degp.reshape(NC, NP, 1), Wp)
    acc = Wp
    out = None
    for l in range(N_LAYERS):
        part = _layer_kernel(y, rowp, colp)
        if l < N_LAYERS - 1:
            y, acc = _tc_layer(part, disb, acc)
        else:
            out = _tc_final(part, disb, acc)
    return out[:N_NODES]


# speedup vs baseline: 16.1566x; 16.1566x over previous
import jax
import jax.numpy as jnp
from jax import lax
from jax.experimental import pallas as pl
from jax.experimental.pallas import tpu as pltpu
from jax.experimental.pallas import tpu_sc as plsc

N_NODES = 50000
DIM = 32
DH = DIM // 2
N_EDGES = 1600000
N_LAYERS = 3

NC, NS = 2, 16
NW = NC * NS

NP = 50048
DUMMY = 2176
NACC = NP + DUMMY

EPT = 51200
E_PAD = EPT * NW
CHUNK = 2048
N_CHUNKS = EPT // CHUNK
JSTEPS = CHUNK // 128
E2D = E_PAD // 128

ZROWS = NACC // NS
ZHALF = ZROWS // 2
DROWS = NP // NS
DHALF = 1568

_MESH = plsc.VectorSubcoreMesh(core_axis_name="c", subcore_axis_name="s")
_SC_PARAMS = pltpu.CompilerParams(use_tc_tiling_on_sc=False)


def _deg_body(colp, degp, colbuf, ones1, zbuf, acc):
    c = lax.axis_index("c")
    s = lax.axis_index("s")
    w = c * NS + s

    def fill(i, _):
        ones1[pl.ds(i * 16, 16)] = jnp.full((16,), 1.0, jnp.float32)
        return 0
    lax.fori_loop(0, 8, fill, 0)

    def fillz(i, _):
        zbuf[pl.ds(i * 16, 16)] = jnp.zeros((16,), jnp.float32)
        return 0
    lax.fori_loop(0, ZROWS // 16, fillz, 0)
    pltpu.sync_copy(zbuf, acc.at[pl.ds(s * ZROWS, ZROWS)])
    plsc.subcore_barrier()

    eb = w * (EPT // 128)

    def chunk(i, _):
        pltpu.sync_copy(colp.at[pl.ds(eb + i * JSTEPS, JSTEPS)], colbuf)
        for j in range(JSTEPS):
            pltpu.sync_copy(ones1, acc.at[colbuf.at[j]], add=True)
        return 0
    lax.fori_loop(0, N_CHUNKS, chunk, 0)
    plsc.subcore_barrier()

    pltpu.sync_copy(acc.at[pl.ds(s * DROWS, DROWS)], zbuf.at[pl.ds(0, DROWS)])
    pltpu.sync_copy(zbuf.at[pl.ds(0, DROWS)],
                    degp.at[pl.ds(c * NP + s * DROWS, DROWS)])


_deg_kernel = pl.kernel(
    _deg_body,
    out_type=jax.ShapeDtypeStruct((NC * NP,), jnp.float32),
    mesh=_MESH,
    compiler_params=_SC_PARAMS,
    scratch_types=[
        pltpu.VMEM((JSTEPS, 128), jnp.int32),
        pltpu.VMEM((128,), jnp.float32),
        pltpu.VMEM((ZROWS,), jnp.float32),
        pltpu.VMEM_SHARED((NACC,), jnp.float32),
    ],
)


def _layer_body(y2, rowp, colp, part, rowbuf, colbuf, rows, acc, sem):
    c = lax.axis_index("c")
    s = lax.axis_index("s")
    w = c * NS + s
    eb = w * (EPT // 128)

    for h in range(2):
        def fillz(i, _):
            rows[i, pl.ds(0, 16)] = jnp.zeros((16,), jnp.float32)
            return 0
        lax.fori_loop(0, ZHALF, fillz, 0)
        for k in range(2):
            pltpu.sync_copy(rows.at[pl.ds(0, ZHALF)],
                            acc.at[pl.ds(s * ZROWS + k * ZHALF, ZHALF)])
        plsc.subcore_barrier()

        def chunk(i, _):
            pltpu.sync_copy(rowp.at[pl.ds(eb + i * JSTEPS, JSTEPS)], rowbuf)
            pltpu.sync_copy(colp.at[pl.ds(eb + i * JSTEPS, JSTEPS)], colbuf)
            descs = [
                pltpu.async_copy(y2.at[h].at[rowbuf.at[j]],
                                 rows.at[pl.ds(j * 128, 128)], sem)
                for j in range(JSTEPS)
            ]
            for d in descs:
                d.wait()
            for j in range(JSTEPS):
                pltpu.sync_copy(rows.at[pl.ds(j * 128, 128)],
                                acc.at[colbuf.at[j]], add=True)
            return 0
        lax.fori_loop(0, N_CHUNKS, chunk, 0)
        plsc.subcore_barrier()

        for off, cnt in ((0, DHALF), (DHALF, DROWS - DHALF)):
            base = s * DROWS + off
            pltpu.sync_copy(acc.at[pl.ds(base, cnt)], rows.at[pl.ds(0, cnt)])
            pltpu.sync_copy(rows.at[pl.ds(0, cnt)],
                            part.at[h, c, pl.ds(base, cnt)])
        plsc.subcore_barrier()


_layer_kernel = pl.kernel(
    _layer_body,
    out_type=jax.ShapeDtypeStruct((2, NC, NP, DH), jnp.float32),
    mesh=_MESH,
    compiler_params=_SC_PARAMS,
    scratch_types=[
        pltpu.VMEM((JSTEPS, 128), jnp.int32),
        pltpu.VMEM((JSTEPS, 128), jnp.int32),
        pltpu.VMEM((CHUNK, DH), jnp.float32),
        pltpu.VMEM_SHARED((NACC, DH), jnp.float32),
        pltpu.SemaphoreType.DMA,
    ],
)


BR = NP // 16


def _combine(p_ref):
    return jnp.concatenate([p_ref[0, 0] + p_ref[0, 1],
                            p_ref[1, 0] + p_ref[1, 1]], axis=1)


def _split_y(y):
    return jnp.stack([y[:, :DH], y[:, DH:]], axis=0)


def _tc_init_body(degp_ref, w_ref, disb_ref, y2_ref):
    deg = degp_ref[0] + degp_ref[1]
    dis = jnp.where(deg > 0, lax.rsqrt(deg), 0.0)
    disb = jnp.broadcast_to(dis, (BR, DIM))
    disb_ref[...] = disb
    y2_ref[...] = _split_y(disb * w_ref[...])


_tc_init = pl.pallas_call(
    _tc_init_body,
    grid=(16,),
    in_specs=[
        pl.BlockSpec((2, BR, 1), lambda i: (0, i, 0)),
        pl.BlockSpec((BR, DIM), lambda i: (i, 0)),
    ],
    out_specs=[
        pl.BlockSpec((BR, DIM), lambda i: (i, 0)),
        pl.BlockSpec((2, BR, DH), lambda i: (0, i, 0)),
    ],
    out_shape=[
        jax.ShapeDtypeStruct((NP, DIM), jnp.float32),
        jax.ShapeDtypeStruct((2, NP, DH), jnp.float32),
    ],
)


def _tc_layer_body(p_ref, disb_ref, acc_ref, y2_ref, accout_ref):
    x = disb_ref[...] * _combine(p_ref)
    accout_ref[...] = acc_ref[...] + x
    y2_ref[...] = _split_y(disb_ref[...] * x)


_tc_layer = pl.pallas_call(
    _tc_layer_body,
    grid=(16,),
    in_specs=[
        pl.BlockSpec((2, 2, BR, DH), lambda i: (0, 0, i, 0)),
        pl.BlockSpec((BR, DIM), lambda i: (i, 0)),
        pl.BlockSpec((BR, DIM), lambda i: (i, 0)),
    ],
    out_specs=[
        pl.BlockSpec((2, BR, DH), lambda i: (0, i, 0)),
        pl.BlockSpec((BR, DIM), lambda i: (i, 0)),
    ],
    out_shape=[
        jax.ShapeDtypeStruct((2, NP, DH), jnp.float32),
        jax.ShapeDtypeStruct((NP, DIM), jnp.float32),
    ],
)


def _tc_final_body(p_ref, disb_ref, acc_ref, out_ref):
    x = disb_ref[...] * _combine(p_ref)
    out_ref[...] = (acc_ref[...] + x) * (1.0 / (N_LAYERS + 1))


_tc_final = pl.pallas_call(
    _tc_final_body,
    grid=(16,),
    in_specs=[
        pl.BlockSpec((2, 2, BR, DH), lambda i: (0, 0, i, 0)),
        pl.BlockSpec((BR, DIM), lambda i: (i, 0)),
        pl.BlockSpec((BR, DIM), lambda i: (i, 0)),
    ],
    out_specs=pl.BlockSpec((BR, DIM), lambda i: (i, 0)),
    out_shape=jax.ShapeDtypeStruct((NP, DIM), jnp.float32),
)


def kernel(edge_index, W):
    row = edge_index[0]
    col = edge_index[1]
    pad = E_PAD - N_EDGES
    rowp = jnp.concatenate(
        [row, jnp.zeros((pad,), jnp.int32)]).reshape(E2D, 128)
    colp = jnp.concatenate(
        [col, NP + (jnp.arange(pad, dtype=jnp.int32) % DUMMY)]).reshape(E2D, 128)
    Wp = jnp.pad(W, ((0, NP - N_NODES), (0, 0)))

    degp = _deg_kernel(colp)
    disb, y2 = _tc_init(degp.reshape(NC, NP, 1), Wp)
    acc = Wp
    out = None
    for l in range(N_LAYERS):
        part = _layer_kernel(y2, rowp, colp)
        if l < N_LAYERS - 1:
            y2, acc = _tc_layer(part, disb, acc)
        else:
            out = _tc_final(part, disb, acc)
    return out[:N_NODES]


# speedup vs baseline: 36.4568x; 2.2565x over previous
import jax
import jax.numpy as jnp
from jax import lax
from jax.experimental import pallas as pl
from jax.experimental.pallas import tpu as pltpu
from jax.experimental.pallas import tpu_sc as plsc

N_NODES = 50000
DIM = 32
DH = DIM // 2
N_EDGES = 1600000
N_LAYERS = 3

NC, NS = 2, 16
NW = NC * NS

NP = 50048
DUMMY = 2176
NACC = NP + DUMMY

EPT = 50048
E_PAD = EPT * NW
REAL_PT = N_EDGES // NW
PAD_PT = EPT - REAL_PT
EB = EPT // 128
E2D = E_PAD // 128

CHUNK = 2048
JSTEPS = CHUNK // 128
FULL_CHUNKS = 24
TAIL_J = 7

ZROWS = NACC // NS
ZHALF = ZROWS // 2
DROWS = NP // NS
DHALF = 1568

_MESH = plsc.VectorSubcoreMesh(core_axis_name="c", subcore_axis_name="s")
_SC_PARAMS = pltpu.CompilerParams(use_tc_tiling_on_sc=False)


def _deg_body(colp, degp, colbuf, ones1, zbuf, acc):
    c = lax.axis_index("c")
    s = lax.axis_index("s")
    w = c * NS + s

    def fill(i, _):
        ones1[pl.ds(i * 16, 16)] = jnp.full((16,), 1.0, jnp.float32)
        return 0
    lax.fori_loop(0, 8, fill, 0)

    def fillz(i, _):
        zbuf[pl.ds(i * 16, 16)] = jnp.zeros((16,), jnp.float32)
        return 0
    lax.fori_loop(0, ZROWS // 16, fillz, 0)
    pltpu.sync_copy(zbuf, acc.at[pl.ds(s * ZROWS, ZROWS)])
    plsc.subcore_barrier()

    eb = w * EB

    def chunk(i, _):
        pltpu.sync_copy(colp.at[pl.ds(eb + i * JSTEPS, JSTEPS)], colbuf)
        for j in range(JSTEPS):
            pltpu.sync_copy(ones1, acc.at[colbuf.at[j]], add=True)
        return 0
    lax.fori_loop(0, FULL_CHUNKS, chunk, 0)
    pltpu.sync_copy(colp.at[pl.ds(eb + FULL_CHUNKS * JSTEPS, TAIL_J)],
                    colbuf.at[pl.ds(0, TAIL_J)])
    for j in range(TAIL_J):
        pltpu.sync_copy(ones1, acc.at[colbuf.at[j]], add=True)
    plsc.subcore_barrier()

    pltpu.sync_copy(acc.at[pl.ds(s * DROWS, DROWS)], zbuf.at[pl.ds(0, DROWS)])
    pltpu.sync_copy(zbuf.at[pl.ds(0, DROWS)],
                    degp.at[pl.ds(c * NP + s * DROWS, DROWS)])


_deg_kernel = pl.kernel(
    _deg_body,
    out_type=jax.ShapeDtypeStruct((NC * NP,), jnp.float32),
    mesh=_MESH,
    compiler_params=_SC_PARAMS,
    scratch_types=[
        pltpu.VMEM((JSTEPS, 128), jnp.int32),
        pltpu.VMEM((128,), jnp.float32),
        pltpu.VMEM((ZROWS,), jnp.float32),
        pltpu.VMEM_SHARED((NACC,), jnp.float32),
    ],
)


def _layer_body(y2, rowp, colp, part, rowbuf, colbuf, rows, acc,
                gsem0, gsem1, ssem0, ssem1):
    c = lax.axis_index("c")
    s = lax.axis_index("s")
    w = c * NS + s
    eb = w * EB
    gsems = (gsem0, gsem1)
    ssems = (ssem0, ssem1)

    def load_idx(b, off, nj=JSTEPS):
        pltpu.sync_copy(rowp.at[pl.ds(off, nj)], rowbuf.at[b, pl.ds(0, nj)])
        pltpu.sync_copy(colp.at[pl.ds(off, nj)], colbuf.at[b, pl.ds(0, nj)])

    def fire_gathers(h, b, nj=JSTEPS):
        for j in range(nj):
            pltpu.async_copy(y2.at[h].at[rowbuf.at[b, j]],
                             rows.at[b, pl.ds(j * 128, 128)], gsems[b])

    def wait_gathers(h, b, nj=JSTEPS):
        for j in range(nj):
            pltpu.make_async_copy(y2.at[h].at[rowbuf.at[b, j]],
                                  rows.at[b, pl.ds(j * 128, 128)],
                                  gsems[b]).wait()

    def scatter(b, nj=JSTEPS):
        descs = [
            pltpu.async_copy(rows.at[b, pl.ds(j * 128, 128)],
                             acc.at[colbuf.at[b, j]], ssems[b], add=True)
            for j in range(nj)
        ]
        for d in descs:
            d.wait()

    for h in range(2):
        def fillz(i, _):
            rows[0, i, pl.ds(0, 16)] = jnp.zeros((16,), jnp.float32)
            return 0
        lax.fori_loop(0, ZHALF, fillz, 0)
        for k in range(2):
            pltpu.sync_copy(rows.at[0, pl.ds(0, ZHALF)],
                            acc.at[pl.ds(s * ZROWS + k * ZHALF, ZHALF)])
        plsc.subcore_barrier()

        load_idx(0, eb)
        fire_gathers(h, 0)
        load_idx(1, eb + JSTEPS)
        fire_gathers(h, 1)

        def steady(k, _):
            base = eb + k * 2 * JSTEPS
            for b in range(2):
                wait_gathers(h, b)
                scatter(b)
                load_idx(b, base + (2 + b) * JSTEPS)
                fire_gathers(h, b)
            return 0
        lax.fori_loop(0, FULL_CHUNKS // 2 - 1, steady, 0)

        for b in range(2):
            wait_gathers(h, b)
            scatter(b)
        load_idx(0, eb + FULL_CHUNKS * JSTEPS, TAIL_J)
        fire_gathers(h, 0, TAIL_J)
        wait_gathers(h, 0, TAIL_J)
        scatter(0, TAIL_J)
        plsc.subcore_barrier()

        for off, cnt in ((0, DHALF), (DHALF, DROWS - DHALF)):
            base = s * DROWS + off
            pltpu.sync_copy(acc.at[pl.ds(base, cnt)],
                            rows.at[0, pl.ds(0, cnt)])
            pltpu.sync_copy(rows.at[0, pl.ds(0, cnt)],
                            part.at[h, c, pl.ds(base, cnt)])
        plsc.subcore_barrier()


_layer_kernel = pl.kernel(
    _layer_body,
    out_type=jax.ShapeDtypeStruct((2, NC, NP, DH), jnp.float32),
    mesh=_MESH,
    compiler_params=_SC_PARAMS,
    scratch_types=[
        pltpu.VMEM((2, JSTEPS, 128), jnp.int32),
        pltpu.VMEM((2, JSTEPS, 128), jnp.int32),
        pltpu.VMEM((2, CHUNK, DH), jnp.float32),
        pltpu.VMEM_SHARED((NACC, DH), jnp.float32),
        pltpu.SemaphoreType.DMA,
        pltpu.SemaphoreType.DMA,
        pltpu.SemaphoreType.DMA,
        pltpu.SemaphoreType.DMA,
    ],
)


BR = NP // 16


def _combine(p_ref):
    return jnp.concatenate([p_ref[0, 0] + p_ref[0, 1],
                            p_ref[1, 0] + p_ref[1, 1]], axis=1)


def _split_y(y):
    return jnp.stack([y[:, :DH], y[:, DH:]], axis=0)


def _tc_init_body(degp_ref, w_ref, disb_ref, y2_ref):
    deg = degp_ref[0] + degp_ref[1]
    dis = jnp.where(deg > 0, lax.rsqrt(deg), 0.0)
    disb = jnp.broadcast_to(dis, (BR, DIM))
    disb_ref[...] = disb
    y2_ref[...] = _split_y(disb * w_ref[...])


_tc_init = pl.pallas_call(
    _tc_init_body,
    grid=(16,),
    in_specs=[
        pl.BlockSpec((2, BR, 1), lambda i: (0, i, 0)),
        pl.BlockSpec((BR, DIM), lambda i: (i, 0)),
    ],
    out_specs=[
        pl.BlockSpec((BR, DIM), lambda i: (i, 0)),
        pl.BlockSpec((2, BR, DH), lambda i: (0, i, 0)),
    ],
    out_shape=[
        jax.ShapeDtypeStruct((NP, DIM), jnp.float32),
        jax.ShapeDtypeStruct((2, NP, DH), jnp.float32),
    ],
)


def _tc_layer_body(p_ref, disb_ref, acc_ref, y2_ref, accout_ref):
    x = disb_ref[...] * _combine(p_ref)
    accout_ref[...] = acc_ref[...] + x
    y2_ref[...] = _split_y(disb_ref[...] * x)


_tc_layer = pl.pallas_call(
    _tc_layer_body,
    grid=(16,),
    in_specs=[
        pl.BlockSpec((2, 2, BR, DH), lambda i: (0, 0, i, 0)),
        pl.BlockSpec((BR, DIM), lambda i: (i, 0)),
        pl.BlockSpec((BR, DIM), lambda i: (i, 0)),
    ],
    out_specs=[
        pl.BlockSpec((2, BR, DH), lambda i: (0, i, 0)),
        pl.BlockSpec((BR, DIM), lambda i: (i, 0)),
    ],
    out_shape=[
        jax.ShapeDtypeStruct((2, NP, DH), jnp.float32),
        jax.ShapeDtypeStruct((NP, DIM), jnp.float32),
    ],
)


def _tc_final_body(p_ref, disb_ref, acc_ref, out_ref):
    x = disb_ref[...] * _combine(p_ref)
    out_ref[...] = (acc_ref[...] + x) * (1.0 / (N_LAYERS + 1))


_tc_final = pl.pallas_call(
    _tc_final_body,
    grid=(16,),
    in_specs=[
        pl.BlockSpec((2, 2, BR, DH), lambda i: (0, 0, i, 0)),
        pl.BlockSpec((BR, DIM), lambda i: (i, 0)),
        pl.BlockSpec((BR, DIM), lambda i: (i, 0)),
    ],
    out_specs=pl.BlockSpec((BR, DIM), lambda i: (i, 0)),
    out_shape=jax.ShapeDtypeStruct((NP, DIM), jnp.float32),
)


def kernel(edge_index, W):
    row = edge_index[0]
    col = edge_index[1]
    rpad = jnp.zeros((NW, PAD_PT), jnp.int32)
    cpad = jnp.broadcast_to(
        NP + (jnp.arange(PAD_PT, dtype=jnp.int32) % DUMMY), (NW, PAD_PT))
    rowp = jnp.concatenate(
        [row.reshape(NW, REAL_PT), rpad], axis=1).reshape(E2D, 128)
    colp = jnp.concatenate(
        [col.reshape(NW, REAL_PT), cpad], axis=1).reshape(E2D, 128)
    Wp = jnp.pad(W, ((0, NP - N_NODES), (0, 0)))

    degp = _deg_kernel(colp)
    disb, y2 = _tc_init(degp.reshape(NC, NP, 1), Wp)
    acc = Wp
    out = None
    for l in range(N_LAYERS):
        part = _layer_kernel(y2, rowp, colp)
        if l < N_LAYERS - 1:
            y2, acc = _tc_layer(part, disb, acc)
        else:
            out = _tc_final(part, disb, acc)
    return out[:N_NODES]


# speedup vs baseline: 56.2894x; 1.5440x over previous
import jax
import jax.numpy as jnp
from jax import lax
from jax.experimental import pallas as pl
from jax.experimental.pallas import tpu as pltpu
from jax.experimental.pallas import tpu_sc as plsc

N_NODES = 50000
DIM = 32
DH = DIM // 2
N_EDGES = 1600000
N_LAYERS = 3

NC, NS = 2, 16
NW = NC * NS

NP = 50048

EBLK = N_EDGES // 128
EBLK_LO = EBLK // NW
HI_TILES = EBLK % NW

CHUNK = 2048
JSTEPS = CHUNK // 128
FULL_CHUNKS = 24
TAIL_HI = 7
TAIL_LO = 6

ZROWS = NP // NS
ZHALF = ZROWS // 2
DROWS = ZROWS
DHALF = 1568

RW = 1568
RW_LAST = NP - 31 * RW
RN_LAST = N_NODES - 31 * RW
RB = 784

_MESH = plsc.VectorSubcoreMesh(core_axis_name="c", subcore_axis_name="s",
                               num_cores=NC, num_subcores=NS)
_SC_PARAMS = pltpu.CompilerParams(use_tc_tiling_on_sc=False)


def _rsqrt16(d):
    bits = lax.bitcast_convert_type(d, jnp.int32)
    x = lax.bitcast_convert_type(jnp.int32(0x5F3759DF) - (bits >> 1),
                                 jnp.float32)
    for _ in range(3):
        x = x * (1.5 - 0.5 * d * x * x)
    return jnp.where(d > 0, x, 0.0)


def _tile_blocks(w):
    base = w * EBLK_LO + jnp.minimum(w, HI_TILES)
    return base, w < HI_TILES


def _deg_body(er, degp, colbuf, ones1, zbuf, acc):
    c = lax.axis_index("c")
    s = lax.axis_index("s")
    w = c * NS + s
    eb, is_hi = _tile_blocks(w)

    def fill(i, _):
        ones1[pl.ds(i * 16, 16)] = jnp.full((16,), 1.0, jnp.float32)
        return 0
    lax.fori_loop(0, 8, fill, 0)

    def fillz(i, _):
        zbuf[pl.ds(i * 16, 16)] = jnp.zeros((16,), jnp.float32)
        return 0
    lax.fori_loop(0, ZROWS // 16, fillz, 0)
    pltpu.sync_copy(zbuf, acc.at[pl.ds(s * ZROWS, ZROWS)])
    plsc.subcore_barrier()

    def chunk(i, _):
        pltpu.sync_copy(er.at[1, pl.ds(eb + i * JSTEPS, JSTEPS)], colbuf)
        for j in range(JSTEPS):
            pltpu.sync_copy(ones1, acc.at[colbuf.at[j]], add=True)
        return 0
    lax.fori_loop(0, FULL_CHUNKS, chunk, 0)

    def tail(nj):
        def f():
            pltpu.sync_copy(er.at[1, pl.ds(eb + FULL_CHUNKS * JSTEPS, nj)],
                            colbuf.at[pl.ds(0, nj)])
            for j in range(nj):
                pltpu.sync_copy(ones1, acc.at[colbuf.at[j]], add=True)
        return f
    pl.when(is_hi)(tail(TAIL_HI))
    pl.when(jnp.logical_not(is_hi))(tail(TAIL_LO))
    plsc.subcore_barrier()

    pltpu.sync_copy(acc.at[pl.ds(s * DROWS, DROWS)], zbuf.at[pl.ds(0, DROWS)])
    pltpu.sync_copy(zbuf.at[pl.ds(0, DROWS)],
                    degp.at[pl.ds(c * NP + s * DROWS, DROWS)])


_deg_kernel = pl.kernel(
    _deg_body,
    out_type=jax.ShapeDtypeStruct((NC * NP,), jnp.float32),
    mesh=_MESH,
    compiler_params=_SC_PARAMS,
    scratch_types=[
        pltpu.VMEM((JSTEPS, 128), jnp.int32),
        pltpu.VMEM((128,), jnp.float32),
        pltpu.VMEM((ZROWS,), jnp.float32),
        pltpu.VMEM_SHARED((NP,), jnp.float32),
    ],
)


def _init_body(degp, W, dis, y2, dgb0, dgb1, dbuf, wbuf, ya, yb):
    c = lax.axis_index("c")
    s = lax.axis_index("s")
    w = c * NS + s
    base = w * RW

    def do(cnt_d, cnt_w):
        def f():
            pltpu.sync_copy(degp.at[pl.ds(base, cnt_d)],
                            dgb0.at[pl.ds(0, cnt_d)])
            pltpu.sync_copy(degp.at[pl.ds(NP + base, cnt_d)],
                            dgb1.at[pl.ds(0, cnt_d)])

            def vstep(i, _):
                d = dgb0[pl.ds(i * 16, 16)] + dgb1[pl.ds(i * 16, 16)]
                dbuf[pl.ds(i * 16, 16)] = _rsqrt16(d)
                return 0
            lax.fori_loop(0, cnt_d // 16, vstep, 0)
            pltpu.sync_copy(dbuf.at[pl.ds(0, cnt_d)],
                            dis.at[pl.ds(base, cnt_d)])

            pltpu.sync_copy(W.at[pl.ds(base, cnt_w)], wbuf.at[pl.ds(0, cnt_w)])

            def rstep(i, _):
                dv16 = dbuf[pl.ds(i * 16, 16)]
                for t in range(16):
                    dv = jnp.broadcast_to(dv16[t], (16,))
                    r = i * 16 + t
                    ya[r, pl.ds(0, 16)] = wbuf[r, pl.ds(0, 16)] * dv
                    yb[r, pl.ds(0, 16)] = wbuf[r, pl.ds(16, 16)] * dv
                return 0
            lax.fori_loop(0, cnt_w // 16, rstep, 0)
            pltpu.sync_copy(ya.at[pl.ds(0, cnt_w)],
                            y2.at[0, pl.ds(base, cnt_w)])
            pltpu.sync_copy(yb.at[pl.ds(0, cnt_w)],
                            y2.at[1, pl.ds(base, cnt_w)])
        return f
    pl.when(w < NW - 1)(do(RW, RW))
    pl.when(w == NW - 1)(do(RW_LAST, RN_LAST))


_init_kernel = pl.kernel(
    _init_body,
    out_type=[
        jax.ShapeDtypeStruct((NP,), jnp.float32),
        jax.ShapeDtypeStruct((2, NP, DH), jnp.float32),
    ],
    mesh=_MESH,
    compiler_params=_SC_PARAMS,
    scratch_types=[
        pltpu.VMEM((RW,), jnp.float32),
        pltpu.VMEM((RW,), jnp.float32),
        pltpu.VMEM((RW,), jnp.float32),
        pltpu.VMEM((RW, DIM), jnp.float32),
        pltpu.VMEM((RW, DH), jnp.float32),
        pltpu.VMEM((RW, DH), jnp.float32),
    ],
)


def _layer_body(y2, er, part, rowbuf, colbuf, rows, acc,
                gsem0, gsem1, ssem0, ssem1):
    c = lax.axis_index("c")
    s = lax.axis_index("s")
    w = c * NS + s
    eb, is_hi = _tile_blocks(w)
    gsems = (gsem0, gsem1)
    ssems = (ssem0, ssem1)

    def load_idx(b, off, nj=JSTEPS):
        pltpu.sync_copy(er.at[0, pl.ds(off, nj)], rowbuf.at[b, pl.ds(0, nj)])
        pltpu.sync_copy(er.at[1, pl.ds(off, nj)], colbuf.at[b, pl.ds(0, nj)])

    def fire_gathers(h, b, nj=JSTEPS):
        for j in range(nj):
            pltpu.async_copy(y2.at[h].at[rowbuf.at[b, j]],
                             rows.at[b, pl.ds(j * 128, 128)], gsems[b])

    def wait_gathers(h, b, nj=JSTEPS):
        for j in range(nj):
            pltpu.make_async_copy(y2.at[h].at[rowbuf.at[b, j]],
                                  rows.at[b, pl.ds(j * 128, 128)],
                                  gsems[b]).wait()

    def scatter(b, nj=JSTEPS):
        descs = [
            pltpu.async_copy(rows.at[b, pl.ds(j * 128, 128)],
                             acc.at[colbuf.at[b, j]], ssems[b], add=True)
            for j in range(nj)
        ]
        for d in descs:
            d.wait()

    for h in range(2):
        def fillz(i, _):
            rows[0, i, pl.ds(0, 16)] = jnp.zeros((16,), jnp.float32)
            return 0
        lax.fori_loop(0, ZHALF, fillz, 0)
        for k in range(2):
            pltpu.sync_copy(rows.at[0, pl.ds(0, ZHALF)],
                            acc.at[pl.ds(s * ZROWS + k * ZHALF, ZHALF)])
        plsc.subcore_barrier()

        load_idx(0, eb)
        fire_gathers(h, 0)
        load_idx(1, eb + JSTEPS)
        fire_gathers(h, 1)

        def steady(k, _):
            base = eb + k * 2 * JSTEPS
            for b in range(2):
                wait_gathers(h, b)
                scatter(b)
                load_idx(b, base + (2 + b) * JSTEPS)
                fire_gathers(h, b)
            return 0
        lax.fori_loop(0, FULL_CHUNKS // 2 - 1, steady, 0)

        for b in range(2):
            wait_gathers(h, b)
            scatter(b)

        def tail(nj):
            def f():
                load_idx(0, eb + FULL_CHUNKS * JSTEPS, nj)
                fire_gathers(h, 0, nj)
                wait_gathers(h, 0, nj)
                scatter(0, nj)
            return f
        pl.when(is_hi)(tail(TAIL_HI))
        pl.when(jnp.logical_not(is_hi))(tail(TAIL_LO))
        plsc.subcore_barrier()

        for off, cnt in ((0, DHALF), (DHALF, DROWS - DHALF)):
            base = s * DROWS + off
            pltpu.sync_copy(acc.at[pl.ds(base, cnt)],
                            rows.at[0, pl.ds(0, cnt)])
            pltpu.sync_copy(rows.at[0, pl.ds(0, cnt)],
                            part.at[h, c, pl.ds(base, cnt)])
        plsc.subcore_barrier()


_layer_kernel = pl.kernel(
    _layer_body,
    out_type=jax.ShapeDtypeStruct((2, NC, NP, DH), jnp.float32),
    mesh=_MESH,
    compiler_params=_SC_PARAMS,
    scratch_types=[
        pltpu.VMEM((2, JSTEPS, 128), jnp.int32),
        pltpu.VMEM((2, JSTEPS, 128), jnp.int32),
        pltpu.VMEM((2, CHUNK, DH), jnp.float32),
        pltpu.VMEM_SHARED((NP, DH), jnp.float32),
        pltpu.SemaphoreType.DMA,
        pltpu.SemaphoreType.DMA,
        pltpu.SemaphoreType.DMA,
        pltpu.SemaphoreType.DMA,
    ],
)


def _comb_body(part, dis, acc, y2o, acco, p0, p1, dbuf, accb):
    c = lax.axis_index("c")
    s = lax.axis_index("s")
    w = c * NS + s
    base = w * RW
    pltpu.sync_copy(dis.at[pl.ds(base, RW)], dbuf)

    def sub(off, cnt):
        def f():
            b2 = base + off
            pltpu.sync_copy(acc.at[pl.ds(b2, cnt)], accb.at[pl.ds(0, cnt)])
            for h in range(2):
                pltpu.sync_copy(part.at[h, 0, pl.ds(b2, cnt)],
                                p0.at[pl.ds(0, cnt)])
                pltpu.sync_copy(part.at[h, 1, pl.ds(b2, cnt)],
                                p1.at[pl.ds(0, cnt)])

                def rstep(i, _):
                    dv16 = dbuf[pl.ds(off + i * 16, 16)]
                    for t in range(16):
                        dv = jnp.broadcast_to(dv16[t], (16,))
                        r = i * 16 + t
                        x = (p0[r, pl.ds(0, 16)] +
                             p1[r, pl.ds(0, 16)]) * dv
                        accb[r, pl.ds(h * 16, 16)] = \
                            accb[r, pl.ds(h * 16, 16)] + x
                        p0[r, pl.ds(0, 16)] = x * dv
                    return 0
                lax.fori_loop(0, cnt // 16, rstep, 0)
                pltpu.sync_copy(p0.at[pl.ds(0, cnt)],
                                y2o.at[h, pl.ds(b2, cnt)])
            pltpu.sync_copy(accb.at[pl.ds(0, cnt)], acco.at[pl.ds(b2, cnt)])
        return f
    sub(0, RB)()
    pl.when(w < NW - 1)(sub(RB, RB))
    pl.when(w == NW - 1)(sub(RB, RW_LAST - RB))


_comb_kernel = pl.kernel(
    _comb_body,
    out_type=[
        jax.ShapeDtypeStruct((2, NP, DH), jnp.float32),
        jax.ShapeDtypeStruct((NP, DIM), jnp.float32),
    ],
    mesh=_MESH,
    compiler_params=_SC_PARAMS,
    scratch_types=[
        pltpu.VMEM((RB, DH), jnp.float32),
        pltpu.VMEM((RB, DH), jnp.float32),
        pltpu.VMEM((RW,), jnp.float32),
        pltpu.VMEM((RB, DIM), jnp.float32),
    ],
)


def _final_body(part, dis, acc, out, p0, p1, dbuf, accb):
    c = lax.axis_index("c")
    s = lax.axis_index("s")
    w = c * NS + s
    base = w * RW
    scale = 1.0 / (N_LAYERS + 1)
    pltpu.sync_copy(dis.at[pl.ds(base, RW)], dbuf)

    def sub(off, cnt):
        def f():
            b2 = base + off
            pltpu.sync_copy(acc.at[pl.ds(b2, cnt)], accb.at[pl.ds(0, cnt)])
            for h in range(2):
                pltpu.sync_copy(part.at[h, 0, pl.ds(b2, cnt)],
                                p0.at[pl.ds(0, cnt)])
                pltpu.sync_copy(part.at[h, 1, pl.ds(b2, cnt)],
                                p1.at[pl.ds(0, cnt)])

                def rstep(i, _):
                    dv16 = dbuf[pl.ds(off + i * 16, 16)]
                    for t in range(16):
                        dv = jnp.broadcast_to(dv16[t], (16,))
                        r = i * 16 + t
                        x = (p0[r, pl.ds(0, 16)] +
                             p1[r, pl.ds(0, 16)]) * dv
                        accb[r, pl.ds(h * 16, 16)] = \
                            (accb[r, pl.ds(h * 16, 16)] + x) * scale
                    return 0
                lax.fori_loop(0, cnt // 16, rstep, 0)
            pltpu.sync_copy(accb.at[pl.ds(0, cnt)], out.at[pl.ds(b2, cnt)])
        return f
    sub(0, RB)()
    pl.when(w < NW - 1)(sub(RB, RB))
    pl.when(w == NW - 1)(sub(RB, RN_LAST - RB))


_final_kernel = pl.kernel(
    _final_body,
    out_type=jax.ShapeDtypeStruct((N_NODES, DIM), jnp.float32),
    mesh=_MESH,
    compiler_params=_SC_PARAMS,
    scratch_types=[
        pltpu.VMEM((RB, DH), jnp.float32),
        pltpu.VMEM((RB, DH), jnp.float32),
        pltpu.VMEM((RW,), jnp.float32),
        pltpu.VMEM((RB, DIM), jnp.float32),
    ],
)


def kernel(edge_index, W):
    er = edge_index.reshape(2, EBLK, 128)
    degp = _deg_kernel(er)
    dis, y2 = _init_kernel(degp, W)
    acc = jnp.pad(W, ((0, NP - N_NODES), (0, 0)))
    for l in range(N_LAYERS):
        part = _layer_kernel(y2, er)
        if l < N_LAYERS - 1:
            y2, acc = _comb_kernel(part, dis, acc)
        else:
            out = _final_kernel(part, dis, acc)
    return out


# speedup vs baseline: 68.0667x; 1.2092x over previous
import jax
import jax.numpy as jnp
from jax import lax
from jax.experimental import pallas as pl
from jax.experimental.pallas import tpu as pltpu
from jax.experimental.pallas import tpu_sc as plsc

N_NODES = 50000
DIM = 32
DH = DIM // 2
N_EDGES = 1600000
N_LAYERS = 3

NC, NS = 2, 16
NW = NC * NS

NP = 50048

EBLK = N_EDGES // 128
EBLK_LO = EBLK // NW
HI_TILES = EBLK % NW

JSTEPS = 16
FULL_CHUNKS = 24
TAIL_HI = 7
TAIL_LO = 6
CHUNK = JSTEPS * 128

LJ = 3
LCHUNK = LJ * 128
LFULL = 130
LTAIL = 1

ZROWS = NP // NS
ZHALF = ZROWS // 2
DROWS = ZROWS
DHALF = 1568

RW = 1568
RW_LAST = NP - 31 * RW
RN_LAST = N_NODES - 31 * RW
RB = 784

_MESH = plsc.VectorSubcoreMesh(core_axis_name="c", subcore_axis_name="s",
                               num_cores=NC, num_subcores=NS)
_SC_PARAMS = pltpu.CompilerParams(use_tc_tiling_on_sc=False)


def _rsqrt16(d):
    bits = lax.bitcast_convert_type(d, jnp.int32)
    x = lax.bitcast_convert_type(jnp.int32(0x5F3759DF) - (bits >> 1),
                                 jnp.float32)
    for _ in range(3):
        x = x * (1.5 - 0.5 * d * x * x)
    return jnp.where(d > 0, x, 0.0)


def _tile_blocks(w):
    base = w * EBLK_LO + jnp.minimum(w, HI_TILES)
    return base, w < HI_TILES


def _deg_body(er, degp, colbuf, ones1, zbuf, acc, ssem0, ssem1):
    c = lax.axis_index("c")
    s = lax.axis_index("s")
    w = c * NS + s
    eb, is_hi = _tile_blocks(w)
    ssems = (ssem0, ssem1)

    def fill(i, _):
        ones1[pl.ds(i * 16, 16)] = jnp.full((16,), 1.0, jnp.float32)
        return 0
    lax.fori_loop(0, 8, fill, 0)

    def fillz(i, _):
        zbuf[pl.ds(i * 16, 16)] = jnp.zeros((16,), jnp.float32)
        return 0
    lax.fori_loop(0, ZROWS // 16, fillz, 0)
    pltpu.sync_copy(zbuf, acc.at[pl.ds(s * ZROWS, ZROWS)])
    plsc.subcore_barrier()

    def load_idx(b, blk, nj=JSTEPS):
        pltpu.sync_copy(er.at[1, pl.ds(blk, nj)],
                        colbuf.at[b, pl.ds(0, nj)])

    def scat_fire(b, nj=JSTEPS):
        for j in range(nj):
            pltpu.async_copy(ones1, acc.at[colbuf.at[b, j]],
                             ssems[b], add=True)

    def scat_drain(b, nj=JSTEPS):
        for j in range(nj):
            pltpu.make_async_copy(
                ones1, acc.at[colbuf.at[b, j]], ssems[b]).wait()

    load_idx(0, eb)
    scat_fire(0)
    load_idx(1, eb + JSTEPS)
    scat_fire(1)

    def steady(k, _):
        base = eb + k * 2 * JSTEPS
        for b in range(2):
            scat_drain(b)
            load_idx(b, base + (2 + b) * JSTEPS)
            scat_fire(b)
        return 0
    lax.fori_loop(0, FULL_CHUNKS // 2 - 1, steady, 0)
    for b in range(2):
        scat_drain(b)

    def tail(nj):
        def f():
            load_idx(0, eb + FULL_CHUNKS * JSTEPS, nj)
            scat_fire(0, nj)
            scat_drain(0, nj)
        return f
    pl.when(is_hi)(tail(TAIL_HI))
    pl.when(jnp.logical_not(is_hi))(tail(TAIL_LO))
    plsc.subcore_barrier()

    pltpu.sync_copy(acc.at[pl.ds(s * DROWS, DROWS)], zbuf.at[pl.ds(0, DROWS)])
    pltpu.sync_copy(zbuf.at[pl.ds(0, DROWS)],
                    degp.at[pl.ds(c * NP + s * DROWS, DROWS)])


_deg_kernel = pl.kernel(
    _deg_body,
    out_type=jax.ShapeDtypeStruct((NC * NP,), jnp.float32),
    mesh=_MESH,
    compiler_params=_SC_PARAMS,
    scratch_types=[
        pltpu.VMEM((2, JSTEPS, 128), jnp.int32),
        pltpu.VMEM((128,), jnp.float32),
        pltpu.VMEM((ZROWS,), jnp.float32),
        pltpu.VMEM_SHARED((NP,), jnp.float32),
        pltpu.SemaphoreType.DMA,
        pltpu.SemaphoreType.DMA,
    ],
)


def _init_body(degp, W, dis, y0, dgb0, dgb1, dbuf, wbuf):
    c = lax.axis_index("c")
    s = lax.axis_index("s")
    w = c * NS + s
    base = w * RW

    def do(cnt_d, cnt_w):
        def f():
            pltpu.sync_copy(degp.at[pl.ds(base, cnt_d)],
                            dgb0.at[pl.ds(0, cnt_d)])
            pltpu.sync_copy(degp.at[pl.ds(NP + base, cnt_d)],
                            dgb1.at[pl.ds(0, cnt_d)])

            def vstep(i, _):
                d = dgb0[pl.ds(i * 16, 16)] + dgb1[pl.ds(i * 16, 16)]
                dbuf[pl.ds(i * 16, 16)] = _rsqrt16(d)
                return 0
            lax.fori_loop(0, cnt_d // 16, vstep, 0)
            pltpu.sync_copy(dbuf.at[pl.ds(0, cnt_d)],
                            dis.at[pl.ds(base, cnt_d)])

            pltpu.sync_copy(W.at[pl.ds(base, cnt_w)], wbuf.at[pl.ds(0, cnt_w)])

            def rstep(i, _):
                dv16 = dbuf[pl.ds(i * 16, 16)]
                for t in range(16):
                    dv = jnp.broadcast_to(dv16[t], (16,))
                    r = i * 16 + t
                    wbuf[r, pl.ds(0, 16)] = wbuf[r, pl.ds(0, 16)] * dv
                    wbuf[r, pl.ds(16, 16)] = wbuf[r, pl.ds(16, 16)] * dv
                return 0
            lax.fori_loop(0, cnt_w // 16, rstep, 0)
            pltpu.sync_copy(wbuf.at[pl.ds(0, cnt_w)],
                            y0.at[pl.ds(base, cnt_w)])
        return f
    pl.when(w < NW - 1)(do(RW, RW))
    pl.when(w == NW - 1)(do(RW_LAST, RN_LAST))


_init_kernel = pl.kernel(
    _init_body,
    out_type=[
        jax.ShapeDtypeStruct((NP,), jnp.float32),
        jax.ShapeDtypeStruct((NP, DIM), jnp.float32),
    ],
    mesh=_MESH,
    compiler_params=_SC_PARAMS,
    scratch_types=[
        pltpu.VMEM((RW,), jnp.float32),
        pltpu.VMEM((RW,), jnp.float32),
        pltpu.VMEM((RW,), jnp.float32),
        pltpu.VMEM((RW, DIM), jnp.float32),
    ],
)


def _layer_body(y, er, part, rowbuf, colbuf, rows, acc,
                gsem0, gsem1, ssem0, ssem1):
    c = lax.axis_index("c")
    s = lax.axis_index("s")
    w = c * NS + s
    eb, is_hi = _tile_blocks(w)
    gsems = (gsem0, gsem1)
    ssems = (ssem0, ssem1)

    def load_rows(b, blk, nj=LJ):
        pltpu.sync_copy(er.at[0, pl.ds(blk, nj)],
                        rowbuf.at[b, pl.ds(0, nj)])

    def load_cols(b, blk, nj=LJ):
        pltpu.sync_copy(er.at[1, pl.ds(blk, nj)],
                        colbuf.at[b, pl.ds(0, nj)])

    def fire_gathers(b, nj=LJ):
        for j in range(nj):
            pltpu.async_copy(y.at[rowbuf.at[b, j]],
                             rows.at[b, pl.ds(j * 128, 128)], gsems[b])

    def wait_gathers(b, nj=LJ):
        for j in range(nj):
            pltpu.make_async_copy(
                y.at[rowbuf.at[b, j]],
                rows.at[b, pl.ds(j * 128, 128)], gsems[b]).wait()

    def scat_fire(b, nj=LJ):
        for j in range(nj):
            pltpu.async_copy(rows.at[b, pl.ds(j * 128, 128)],
                             acc.at[colbuf.at[b, j]], ssems[b], add=True)

    def scat_drain(b, nj=LJ):
        for j in range(nj):
            pltpu.make_async_copy(
                rows.at[b, pl.ds(j * 128, 128)],
                acc.at[colbuf.at[b, j]], ssems[b]).wait()

    def fillz(i, _):
        rows[0, i, pl.ds(0, 16)] = jnp.zeros((16,), jnp.float32)
        rows[0, i, pl.ds(16, 16)] = jnp.zeros((16,), jnp.float32)
        return 0
    lax.fori_loop(0, LCHUNK, fillz, 0)
    for k in range(8):
        pltpu.sync_copy(rows.at[0],
                        acc.at[pl.ds(s * ZROWS + k * LCHUNK, LCHUNK)])
    pltpu.sync_copy(rows.at[0, pl.ds(0, ZROWS - 8 * LCHUNK)],
                    acc.at[pl.ds(s * ZROWS + 8 * LCHUNK,
                                 ZROWS - 8 * LCHUNK)])
    plsc.subcore_barrier()

    load_rows(0, eb)
    load_cols(0, eb)
    fire_gathers(0)
    load_rows(1, eb + LJ)
    load_cols(1, eb + LJ)
    fire_gathers(1)

    def steady(k, _):
        base = eb + k * 2 * LJ
        for b in range(2):
            wait_gathers(b)
            scat_fire(b)
            load_rows(b, base + (2 + b) * LJ)
            scat_drain(b)
            load_cols(b, base + (2 + b) * LJ)
            fire_gathers(b)
        return 0
    lax.fori_loop(0, LFULL // 2 - 1, steady, 0)

    for b in range(2):
        wait_gathers(b)
        scat_fire(b)
        scat_drain(b)

    def tail():
        load_rows(0, eb + LFULL * LJ, LTAIL)
        load_cols(0, eb + LFULL * LJ, LTAIL)
        fire_gathers(0, LTAIL)
        wait_gathers(0, LTAIL)
        scat_fire(0, LTAIL)
        scat_drain(0, LTAIL)
    pl.when(is_hi)(tail)
    plsc.subcore_barrier()

    nhop = ZROWS // LCHUNK + 1
    for k in range(nhop):
        base = s * DROWS + k * LCHUNK
        cnt = LCHUNK if k < nhop - 1 else DROWS - (nhop - 1) * LCHUNK
        pltpu.sync_copy(acc.at[pl.ds(base, cnt)],
                        rows.at[0, pl.ds(0, cnt)])
        pltpu.sync_copy(rows.at[0, pl.ds(0, cnt)],
                        part.at[c, pl.ds(base, cnt)])
    plsc.subcore_barrier()


_layer_kernel = pl.kernel(
    _layer_body,
    out_type=jax.ShapeDtypeStruct((NC, NP, DIM), jnp.float32),
    mesh=_MESH,
    compiler_params=_SC_PARAMS,
    scratch_types=[
        pltpu.VMEM((2, LJ, 128), jnp.int32),
        pltpu.VMEM((2, LJ, 128), jnp.int32),
        pltpu.VMEM((2, LCHUNK, DIM), jnp.float32),
        pltpu.VMEM_SHARED((NP, DIM), jnp.float32),
        pltpu.SemaphoreType.DMA,
        pltpu.SemaphoreType.DMA,
        pltpu.SemaphoreType.DMA,
        pltpu.SemaphoreType.DMA,
    ],
)


def _comb_body(part, dis, acc, yo, acco, p0, p1, dbuf, accb):
    c = lax.axis_index("c")
    s = lax.axis_index("s")
    w = c * NS + s
    base = w * RW
    pltpu.sync_copy(dis.at[pl.ds(base, RW)], dbuf)

    def sub(off, cnt):
        def f():
            b2 = base + off
            pltpu.sync_copy(acc.at[pl.ds(b2, cnt)], accb.at[pl.ds(0, cnt)])
            pltpu.sync_copy(part.at[0, pl.ds(b2, cnt)], p0.at[pl.ds(0, cnt)])
            pltpu.sync_copy(part.at[1, pl.ds(b2, cnt)], p1.at[pl.ds(0, cnt)])

            def rstep(i, _):
                dv16 = dbuf[pl.ds(off + i * 16, 16)]
                for t in range(16):
                    dv = jnp.broadcast_to(dv16[t], (16,))
                    r = i * 16 + t
                    for q in (0, 16):
                        x = (p0[r, pl.ds(q, 16)] +
                             p1[r, pl.ds(q, 16)]) * dv
                        accb[r, pl.ds(q, 16)] = accb[r, pl.ds(q, 16)] + x
                        p0[r, pl.ds(q, 16)] = x * dv
                return 0
            lax.fori_loop(0, cnt // 16, rstep, 0)
            pltpu.sync_copy(p0.at[pl.ds(0, cnt)], yo.at[pl.ds(b2, cnt)])
            pltpu.sync_copy(accb.at[pl.ds(0, cnt)], acco.at[pl.ds(b2, cnt)])
        return f
    sub(0, RB)()
    pl.when(w < NW - 1)(sub(RB, RB))
    pl.when(w == NW - 1)(sub(RB, RW_LAST - RB))


_comb_kernel = pl.kernel(
    _comb_body,
    out_type=[
        jax.ShapeDtypeStruct((NP, DIM), jnp.float32),
        jax.ShapeDtypeStruct((NP, DIM), jnp.float32),
    ],
    mesh=_MESH,
    compiler_params=_SC_PARAMS,
    scratch_types=[
        pltpu.VMEM((RB, DIM), jnp.float32),
        pltpu.VMEM((RB, DIM), jnp.float32),
        pltpu.VMEM((RW,), jnp.float32),
        pltpu.VMEM((RB, DIM), jnp.float32),
    ],
)


def _final_body(part, dis, acc, out, p0, p1, dbuf, accb):
    c = lax.axis_index("c")
    s = lax.axis_index("s")
    w = c * NS + s
    base = w * RW
    scale = 1.0 / (N_LAYERS + 1)
    pltpu.sync_copy(dis.at[pl.ds(base, RW)], dbuf)

    def sub(off, cnt):
        def f():
            b2 = base + off
            pltpu.sync_copy(acc.at[pl.ds(b2, cnt)], accb.at[pl.ds(0, cnt)])
            pltpu.sync_copy(part.at[0, pl.ds(b2, cnt)], p0.at[pl.ds(0, cnt)])
            pltpu.sync_copy(part.at[1, pl.ds(b2, cnt)], p1.at[pl.ds(0, cnt)])

            def rstep(i, _):
                dv16 = dbuf[pl.ds(off + i * 16, 16)]
                for t in range(16):
                    dv = jnp.broadcast_to(dv16[t], (16,))
                    r = i * 16 + t
                    for q in (0, 16):
                        x = (p0[r, pl.ds(q, 16)] +
                             p1[r, pl.ds(q, 16)]) * dv
                        accb[r, pl.ds(q, 16)] = \
                            (accb[r, pl.ds(q, 16)] + x) * scale
                return 0
            lax.fori_loop(0, cnt // 16, rstep, 0)
            pltpu.sync_copy(accb.at[pl.ds(0, cnt)], out.at[pl.ds(b2, cnt)])
        return f
    sub(0, RB)()
    pl.when(w < NW - 1)(sub(RB, RB))
    pl.when(w == NW - 1)(sub(RB, RN_LAST - RB))


_final_kernel = pl.kernel(
    _final_body,
    out_type=jax.ShapeDtypeStruct((N_NODES, DIM), jnp.float32),
    mesh=_MESH,
    compiler_params=_SC_PARAMS,
    scratch_types=[
        pltpu.VMEM((RB, DIM), jnp.float32),
        pltpu.VMEM((RB, DIM), jnp.float32),
        pltpu.VMEM((RW,), jnp.float32),
        pltpu.VMEM((RB, DIM), jnp.float32),
    ],
)


def kernel(edge_index, W):
    er = edge_index.reshape(2, EBLK, 128)
    degp = _deg_kernel(er)
    dis, y = _init_kernel(degp, W)
    acc = jnp.pad(W, ((0, NP - N_NODES), (0, 0)))
    for l in range(N_LAYERS):
        part = _layer_kernel(y, er)
        if l < N_LAYERS - 1:
            y, acc = _comb_kernel(part, dis, acc)
        else:
            out = _final_kernel(part, dis, acc)
    return out


# speedup vs baseline: 68.1459x; 1.0012x over previous
import jax
import jax.numpy as jnp
from jax import lax
from jax.experimental import pallas as pl
from jax.experimental.pallas import tpu as pltpu
from jax.experimental.pallas import tpu_sc as plsc

N_NODES = 50000
DIM = 32
DH = DIM // 2
N_EDGES = 1600000
N_LAYERS = 3

NC, NS = 2, 16
NW = NC * NS

NP = 50048

EBLK = N_EDGES // 128
EBLK_LO = EBLK // NW
HI_TILES = EBLK % NW

JSTEPS = 16
FULL_CHUNKS = 24
TAIL_HI = 7
TAIL_LO = 6
CHUNK = JSTEPS * 128

LJ = 3
LCHUNK = LJ * 128
LFULL = 130
LTAIL = 1

ZROWS = NP // NS
DROWS = ZROWS

RW = 1568
RW_LAST = NP - 31 * RW
RN_LAST = N_NODES - 31 * RW
RB = 784

_MESH = plsc.VectorSubcoreMesh(core_axis_name="c", subcore_axis_name="s",
                               num_cores=NC, num_subcores=NS)
_SC_PARAMS = pltpu.CompilerParams(use_tc_tiling_on_sc=False)


def _rsqrt16(d):
    bits = lax.bitcast_convert_type(d, jnp.int32)
    x = lax.bitcast_convert_type(jnp.int32(0x5F3759DF) - (bits >> 1),
                                 jnp.float32)
    for _ in range(3):
        x = x * (1.5 - 0.5 * d * x * x)
    return jnp.where(d > 0, x, 0.0)


def _tile_blocks(w):
    base = w * EBLK_LO + jnp.minimum(w, HI_TILES)
    return base, w < HI_TILES


def _deg_body(er, degp, colbuf, ones1, zbuf, acc, ssem0, ssem1):
    c = lax.axis_index("c")
    s = lax.axis_index("s")
    w = c * NS + s
    eb, is_hi = _tile_blocks(w)
    ssems = (ssem0, ssem1)

    def fill(i, _):
        ones1[pl.ds(i * 16, 16)] = jnp.full((16,), 1.0, jnp.float32)
        return 0
    lax.fori_loop(0, 8, fill, 0)

    def fillz(i, _):
        zbuf[pl.ds(i * 16, 16)] = jnp.zeros((16,), jnp.float32)
        return 0
    lax.fori_loop(0, ZROWS // 16, fillz, 0)
    pltpu.sync_copy(zbuf, acc.at[pl.ds(s * ZROWS, ZROWS)])
    plsc.subcore_barrier()

    def load_idx(b, blk, nj=JSTEPS):
        pltpu.sync_copy(er.at[1, pl.ds(blk, nj)],
                        colbuf.at[b, pl.ds(0, nj)])

    def scat_fire(b, nj=JSTEPS):
        for j in range(nj):
            pltpu.async_copy(ones1, acc.at[colbuf.at[b, j]],
                             ssems[b], add=True)

    def scat_drain(b, nj=JSTEPS):
        for j in range(nj):
            pltpu.make_async_copy(
                ones1, acc.at[colbuf.at[b, j]], ssems[b]).wait()

    load_idx(0, eb)
    scat_fire(0)
    load_idx(1, eb + JSTEPS)
    scat_fire(1)

    def steady(k, _):
        base = eb + k * 2 * JSTEPS
        for b in range(2):
            scat_drain(b)
            load_idx(b, base + (2 + b) * JSTEPS)
            scat_fire(b)
        return 0
    lax.fori_loop(0, FULL_CHUNKS // 2 - 1, steady, 0)
    for b in range(2):
        scat_drain(b)

    def tail(nj):
        def f():
            load_idx(0, eb + FULL_CHUNKS * JSTEPS, nj)
            scat_fire(0, nj)
            scat_drain(0, nj)
        return f
    pl.when(is_hi)(tail(TAIL_HI))
    pl.when(jnp.logical_not(is_hi))(tail(TAIL_LO))
    plsc.subcore_barrier()

    pltpu.sync_copy(acc.at[pl.ds(s * DROWS, DROWS)], zbuf.at[pl.ds(0, DROWS)])
    pltpu.sync_copy(zbuf.at[pl.ds(0, DROWS)],
                    degp.at[pl.ds(c * NP + s * DROWS, DROWS)])


_deg_kernel = pl.kernel(
    _deg_body,
    out_type=jax.ShapeDtypeStruct((NC * NP,), jnp.float32),
    mesh=_MESH,
    compiler_params=_SC_PARAMS,
    scratch_types=[
        pltpu.VMEM((2, JSTEPS, 128), jnp.int32),
        pltpu.VMEM((128,), jnp.float32),
        pltpu.VMEM((ZROWS,), jnp.float32),
        pltpu.VMEM_SHARED((NP,), jnp.float32),
        pltpu.SemaphoreType.DMA,
        pltpu.SemaphoreType.DMA,
    ],
)


def _init_body(degp, W, dis, y0, dgb0, dgb1, dbuf, wbuf):
    c = lax.axis_index("c")
    s = lax.axis_index("s")
    w = c * NS + s
    base = w * RW

    def do(cnt_d, cnt_w):
        def f():
            pltpu.sync_copy(degp.at[pl.ds(base, cnt_d)],
                            dgb0.at[pl.ds(0, cnt_d)])
            pltpu.sync_copy(degp.at[pl.ds(NP + base, cnt_d)],
                            dgb1.at[pl.ds(0, cnt_d)])

            def vstep(i, _):
                d = dgb0[pl.ds(i * 16, 16)] + dgb1[pl.ds(i * 16, 16)]
                dbuf[pl.ds(i * 16, 16)] = _rsqrt16(d)
                return 0
            lax.fori_loop(0, cnt_d // 16, vstep, 0)
            pltpu.sync_copy(dbuf.at[pl.ds(0, cnt_d)],
                            dis.at[pl.ds(base, cnt_d)])

            pltpu.sync_copy(W.at[pl.ds(base, cnt_w)], wbuf.at[pl.ds(0, cnt_w)])

            def rstep(i, _):
                dv16 = dbuf[pl.ds(i * 16, 16)]
                for t in range(16):
                    dv = jnp.broadcast_to(dv16[t], (16,))
                    r = i * 16 + t
                    wbuf[r, pl.ds(0, 16)] = wbuf[r, pl.ds(0, 16)] * dv
                    wbuf[r, pl.ds(16, 16)] = wbuf[r, pl.ds(16, 16)] * dv
                return 0
            lax.fori_loop(0, cnt_w // 16, rstep, 0)
            pltpu.sync_copy(wbuf.at[pl.ds(0, cnt_w)],
                            y0.at[pl.ds(base, cnt_w)])
        return f
    pl.when(w < NW - 1)(do(RW, RW))
    pl.when(w == NW - 1)(do(RW_LAST, RN_LAST))


_init_kernel = pl.kernel(
    _init_body,
    out_type=[
        jax.ShapeDtypeStruct((NP,), jnp.float32),
        jax.ShapeDtypeStruct((NP, DIM), jnp.float32),
    ],
    mesh=_MESH,
    compiler_params=_SC_PARAMS,
    scratch_types=[
        pltpu.VMEM((RW,), jnp.float32),
        pltpu.VMEM((RW,), jnp.float32),
        pltpu.VMEM((RW,), jnp.float32),
        pltpu.VMEM((RW, DIM), jnp.float32),
    ],
)


def _layer_body(y, er, part, rowbuf, colbuf, rows, acc,
                gsem0, gsem1, ssem0, ssem1):
    c = lax.axis_index("c")
    s = lax.axis_index("s")
    w = c * NS + s
    eb, is_hi = _tile_blocks(w)
    gsems = (gsem0, gsem1)
    ssems = (ssem0, ssem1)

    def load_rows(b, blk, nj=LJ):
        pltpu.sync_copy(er.at[0, pl.ds(blk, nj)],
                        rowbuf.at[b, pl.ds(0, nj)])

    def load_cols(b, blk, nj=LJ):
        pltpu.sync_copy(er.at[1, pl.ds(blk, nj)],
                        colbuf.at[b, pl.ds(0, nj)])

    def fire_gathers(b, nj=LJ):
        for j in range(nj):
            pltpu.async_copy(y.at[rowbuf.at[b, j]],
                             rows.at[b, pl.ds(j * 128, 128)], gsems[b])

    def wait_gathers(b, nj=LJ):
        for j in range(nj):
            pltpu.make_async_copy(
                y.at[rowbuf.at[b, j]],
                rows.at[b, pl.ds(j * 128, 128)], gsems[b]).wait()

    def scat_fire(b, nj=LJ):
        for j in range(nj):
            pltpu.async_copy(rows.at[b, pl.ds(j * 128, 128)],
                             acc.at[colbuf.at[b, j]], ssems[b], add=True)

    def scat_drain(b, nj=LJ):
        for j in range(nj):
            pltpu.make_async_copy(
                rows.at[b, pl.ds(j * 128, 128)],
                acc.at[colbuf.at[b, j]], ssems[b]).wait()

    def fillz(i, _):
        rows[0, i, pl.ds(0, 16)] = jnp.zeros((16,), jnp.float32)
        rows[0, i, pl.ds(16, 16)] = jnp.zeros((16,), jnp.float32)
        return 0
    lax.fori_loop(0, LCHUNK, fillz, 0)
    for k in range(8):
        pltpu.sync_copy(rows.at[0],
                        acc.at[pl.ds(s * ZROWS + k * LCHUNK, LCHUNK)])
    pltpu.sync_copy(rows.at[0, pl.ds(0, ZROWS - 8 * LCHUNK)],
                    acc.at[pl.ds(s * ZROWS + 8 * LCHUNK,
                                 ZROWS - 8 * LCHUNK)])
    plsc.subcore_barrier()

    load_rows(0, eb)
    load_cols(0, eb)
    fire_gathers(0)
    load_rows(1, eb + LJ)
    load_cols(1, eb + LJ)
    fire_gathers(1)

    def steady(k, _):
        base = eb + k * 2 * LJ
        for b in range(2):
            wait_gathers(b)
            scat_fire(b)
            load_rows(b, base + (2 + b) * LJ)
            scat_drain(b)
            load_cols(b, base + (2 + b) * LJ)
            fire_gathers(b)
        return 0
    lax.fori_loop(0, LFULL // 2 - 1, steady, 0)

    for b in range(2):
        wait_gathers(b)
        scat_fire(b)
        scat_drain(b)

    def tail():
        load_rows(0, eb + LFULL * LJ, LTAIL)
        load_cols(0, eb + LFULL * LJ, LTAIL)
        fire_gathers(0, LTAIL)
        wait_gathers(0, LTAIL)
        scat_fire(0, LTAIL)
        scat_drain(0, LTAIL)
    pl.when(is_hi)(tail)
    plsc.subcore_barrier()

    nhop = ZROWS // LCHUNK + 1
    for k in range(nhop):
        base = s * DROWS + k * LCHUNK
        cnt = LCHUNK if k < nhop - 1 else DROWS - (nhop - 1) * LCHUNK
        pltpu.sync_copy(acc.at[pl.ds(base, cnt)],
                        rows.at[0, pl.ds(0, cnt)])
        pltpu.sync_copy(rows.at[0, pl.ds(0, cnt)],
                        part.at[c, pl.ds(base, cnt)])
    plsc.subcore_barrier()


_layer_kernel = pl.kernel(
    _layer_body,
    out_type=jax.ShapeDtypeStruct((NC, NP, DIM), jnp.float32),
    mesh=_MESH,
    compiler_params=_SC_PARAMS,
    scratch_types=[
        pltpu.VMEM((2, LJ, 128), jnp.int32),
        pltpu.VMEM((2, LJ, 128), jnp.int32),
        pltpu.VMEM((2, LCHUNK, DIM), jnp.float32),
        pltpu.VMEM_SHARED((NP, DIM), jnp.float32),
        pltpu.SemaphoreType.DMA,
        pltpu.SemaphoreType.DMA,
        pltpu.SemaphoreType.DMA,
        pltpu.SemaphoreType.DMA,
    ],
)


def _comb_body(part, dis, acc, yo, acco, p0, p1, dbuf, accb):
    c = lax.axis_index("c")
    s = lax.axis_index("s")
    w = c * NS + s
    base = w * RW
    pltpu.sync_copy(dis.at[pl.ds(base, RW)], dbuf)

    def sub(off, cnt):
        def f():
            b2 = base + off
            pltpu.sync_copy(acc.at[pl.ds(b2, cnt)], accb.at[pl.ds(0, cnt)])
            pltpu.sync_copy(part.at[0, pl.ds(b2, cnt)], p0.at[pl.ds(0, cnt)])
            pltpu.sync_copy(part.at[1, pl.ds(b2, cnt)], p1.at[pl.ds(0, cnt)])

            def rstep(i, _):
                dv16 = dbuf[pl.ds(off + i * 16, 16)]
                for t in range(16):
                    dv = jnp.broadcast_to(dv16[t], (16,))
                    r = i * 16 + t
                    for q in (0, 16):
                        x = (p0[r, pl.ds(q, 16)] +
                             p1[r, pl.ds(q, 16)]) * dv
                        accb[r, pl.ds(q, 16)] = accb[r, pl.ds(q, 16)] + x
                        p0[r, pl.ds(q, 16)] = x * dv
                return 0
            lax.fori_loop(0, cnt // 16, rstep, 0)
            pltpu.sync_copy(p0.at[pl.ds(0, cnt)], yo.at[pl.ds(b2, cnt)])
            pltpu.sync_copy(accb.at[pl.ds(0, cnt)], acco.at[pl.ds(b2, cnt)])
        return f
    sub(0, RB)()
    pl.when(w < NW - 1)(sub(RB, RB))
    pl.when(w == NW - 1)(sub(RB, RW_LAST - RB))


_comb_kernel = pl.kernel(
    _comb_body,
    out_type=[
        jax.ShapeDtypeStruct((NP, DIM), jnp.float32),
        jax.ShapeDtypeStruct((NP, DIM), jnp.float32),
    ],
    mesh=_MESH,
    compiler_params=_SC_PARAMS,
    scratch_types=[
        pltpu.VMEM((RB, DIM), jnp.float32),
        pltpu.VMEM((RB, DIM), jnp.float32),
        pltpu.VMEM((RW,), jnp.float32),
        pltpu.VMEM((RB, DIM), jnp.float32),
    ],
)


def _final_body(part, dis, acc, out, p0, p1, dbuf, accb):
    c = lax.axis_index("c")
    s = lax.axis_index("s")
    w = c * NS + s
    base = w * RW
    scale = 1.0 / (N_LAYERS + 1)
    pltpu.sync_copy(dis.at[pl.ds(base, RW)], dbuf)

    def sub(off, cnt):
        def f():
            b2 = base + off
            pltpu.sync_copy(acc.at[pl.ds(b2, cnt)], accb.at[pl.ds(0, cnt)])
            pltpu.sync_copy(part.at[0, pl.ds(b2, cnt)], p0.at[pl.ds(0, cnt)])
            pltpu.sync_copy(part.at[1, pl.ds(b2, cnt)], p1.at[pl.ds(0, cnt)])

            def rstep(i, _):
                dv16 = dbuf[pl.ds(off + i * 16, 16)]
                for t in range(16):
                    dv = jnp.broadcast_to(dv16[t], (16,))
                    r = i * 16 + t
                    for q in (0, 16):
                        x = (p0[r, pl.ds(q, 16)] +
                             p1[r, pl.ds(q, 16)]) * dv
                        accb[r, pl.ds(q, 16)] = \
                            (accb[r, pl.ds(q, 16)] + x) * scale
                return 0
            lax.fori_loop(0, cnt // 16, rstep, 0)
            pltpu.sync_copy(accb.at[pl.ds(0, cnt)], out.at[pl.ds(b2, cnt)])
        return f
    sub(0, RB)()
    pl.when(w < NW - 1)(sub(RB, RB))
    pl.when(w == NW - 1)(sub(RB, RN_LAST - RB))


_final_kernel = pl.kernel(
    _final_body,
    out_type=jax.ShapeDtypeStruct((N_NODES, DIM), jnp.float32),
    mesh=_MESH,
    compiler_params=_SC_PARAMS,
    scratch_types=[
        pltpu.VMEM((RB, DIM), jnp.float32),
        pltpu.VMEM((RB, DIM), jnp.float32),
        pltpu.VMEM((RW,), jnp.float32),
        pltpu.VMEM((RB, DIM), jnp.float32),
    ],
)


def kernel(edge_index, W):
    er = edge_index.reshape(2, EBLK, 128)
    degp = _deg_kernel(er)
    dis, y = _init_kernel(degp, W)
    acc = jnp.pad(W, ((0, NP - N_NODES), (0, 0)))
    for l in range(N_LAYERS):
        part = _layer_kernel(y, er)
        if l < N_LAYERS - 1:
            y, acc = _comb_kernel(part, dis, acc)
        else:
            out = _final_kernel(part, dis, acc)
    return out
